# baseline (speedup 1.0000x reference)
import functools

import jax
import jax.numpy as jnp
from jax import lax
from jax.experimental import pallas as pl
from jax.experimental.pallas import tpu as pltpu
from jax.experimental.pallas import tpu_sc as plsc

VOCAB = 100000
SEQ = 2048
BATCH = 4
HID = 768
EPS = 1e-12
N = BATCH * SEQ
NW = 32
TOK_PER_W = N // NW
CH = 64
NBUF = 2
NCH = TOK_PER_W // CH


def _make_sc_gather():
    mesh = plsc.VectorSubcoreMesh(core_axis_name="c", subcore_axis_name="s")

    @functools.partial(
        pl.kernel,
        out_type=jax.ShapeDtypeStruct((N, HID), jnp.float32),
        mesh=mesh,
        scratch_types=[
            pltpu.VMEM((TOK_PER_W,), jnp.int32),
            pltpu.VMEM((NBUF, CH, HID), jnp.float32),
        ] + [pltpu.SemaphoreType.DMA] * NBUF,
    )
    def gather_k(ids_hbm, table_hbm, out_hbm, idx_v, rows_v, *sems):
        wid = lax.axis_index("s") * 2 + lax.axis_index("c")
        base = wid * TOK_PER_W
        pltpu.sync_copy(ids_hbm.at[pl.ds(base, TOK_PER_W)], idx_v)

        def start(c):
            return pltpu.async_copy(
                table_hbm.at[idx_v.at[pl.ds(c * CH, CH)]],
                rows_v.at[c % NBUF], sems[c % NBUF])

        copies = [start(c) for c in range(NBUF - 1)]
        for c in range(NCH):
            if c + NBUF - 1 < NCH:
                copies.append(start(c + NBUF - 1))
            copies[c].wait()
            pltpu.sync_copy(rows_v.at[c % NBUF],
                            out_hbm.at[pl.ds(base + c * CH, CH)])

    return gather_k


_sc_gather = _make_sc_gather()

ROWS = 2048
SBLK = SEQ // ROWS


def _ln_body(x_ref, pos_ref, tt_ref, ttemb_ref, gamma_ref, beta_ref, o_ref):
    x = x_ref[...] + pos_ref[...]
    ttf = tt_ref[...]
    ttv = ttemb_ref[0:1, :] + ttf * (ttemb_ref[1:2, :] - ttemb_ref[0:1, :])
    x = x + ttv
    mean = jnp.mean(x, axis=-1, keepdims=True)
    xc = x - mean
    var = jnp.mean(xc * xc, axis=-1, keepdims=True)
    xn = xc * lax.rsqrt(var + EPS)
    o_ref[...] = xn * gamma_ref[...][None, :] + beta_ref[...][None, :]


_ln_call = pl.pallas_call(
    _ln_body,
    grid=(SBLK, BATCH),
    in_specs=[
        pl.BlockSpec((ROWS, HID), lambda i, j: (j * SBLK + i, 0)),
        pl.BlockSpec((ROWS, HID), lambda i, j: (i, 0)),
        pl.BlockSpec((ROWS, 1), lambda i, j: (j * SBLK + i, 0)),
        pl.BlockSpec((8, HID), lambda i, j: (0, 0)),
        pl.BlockSpec((HID,), lambda i, j: (0,)),
        pl.BlockSpec((HID,), lambda i, j: (0,)),
    ],
    out_specs=pl.BlockSpec((ROWS, HID), lambda i, j: (j * SBLK + i, 0)),
    out_shape=jax.ShapeDtypeStruct((N, HID), jnp.float32),
)


def kernel(input_ids, token_type_ids, word_embeddings, token_type_embeddings,
           position_embeddings, ln_gamma, ln_beta):
    ids = input_ids.reshape(N).astype(jnp.int32)
    tts = token_type_ids.reshape(N, 1).astype(jnp.float32)
    ttemb = jnp.concatenate(
        [token_type_embeddings,
         jnp.zeros((6, HID), token_type_embeddings.dtype)], axis=0)
    x = _sc_gather(ids, word_embeddings)
    out = _ln_call(x, position_embeddings, tts, ttemb, ln_gamma, ln_beta)
    return out.reshape(BATCH, SEQ, HID)


# speedup vs baseline: 1.0138x; 1.0138x over previous
import functools

import jax
import jax.numpy as jnp
from jax import lax
from jax.experimental import pallas as pl
from jax.experimental.pallas import tpu as pltpu
from jax.experimental.pallas import tpu_sc as plsc

VOCAB = 100000
SEQ = 2048
BATCH = 4
HID = 768
EPS = 1e-12
N = BATCH * SEQ
NW = 32
TOK_PER_W = N // NW
CH = 32
NBUF = 4
NCH = TOK_PER_W // CH


def _make_sc_gather():
    mesh = plsc.VectorSubcoreMesh(core_axis_name="c", subcore_axis_name="s")

    @functools.partial(
        pl.kernel,
        out_type=jax.ShapeDtypeStruct((N, HID), jnp.float32),
        mesh=mesh,
        scratch_types=[
            pltpu.VMEM((TOK_PER_W,), jnp.int32),
            pltpu.VMEM((NBUF, CH, HID), jnp.float32),
        ] + [pltpu.SemaphoreType.DMA] * NBUF,
    )
    def gather_k(ids_hbm, table_hbm, out_hbm, idx_v, rows_v, *sems):
        wid = lax.axis_index("s") * 2 + lax.axis_index("c")
        base = wid * TOK_PER_W
        pltpu.sync_copy(ids_hbm.at[pl.ds(base, TOK_PER_W)], idx_v)

        def start(c):
            return pltpu.async_copy(
                table_hbm.at[idx_v.at[pl.ds(c * CH, CH)]],
                rows_v.at[c % NBUF], sems[c % NBUF])

        copies = [start(c) for c in range(NBUF - 1)]
        for c in range(NCH):
            if c + NBUF - 1 < NCH:
                copies.append(start(c + NBUF - 1))
            copies[c].wait()
            pltpu.sync_copy(rows_v.at[c % NBUF],
                            out_hbm.at[pl.ds(base + c * CH, CH)])

    return gather_k


_sc_gather = _make_sc_gather()

ROWS = 2048
SBLK = SEQ // ROWS


def _ln_body(x_ref, pos_ref, tt_ref, ttemb_ref, gamma_ref, beta_ref, o_ref):
    x = x_ref[...] + pos_ref[...]
    ttf = tt_ref[...]
    ttv = ttemb_ref[0:1, :] + ttf * (ttemb_ref[1:2, :] - ttemb_ref[0:1, :])
    x = x + ttv
    mean = jnp.mean(x, axis=-1, keepdims=True)
    xc = x - mean
    var = jnp.mean(xc * xc, axis=-1, keepdims=True)
    xn = xc * lax.rsqrt(var + EPS)
    o_ref[...] = xn * gamma_ref[...][None, :] + beta_ref[...][None, :]


_ln_call = pl.pallas_call(
    _ln_body,
    grid=(SBLK, BATCH),
    in_specs=[
        pl.BlockSpec((ROWS, HID), lambda i, j: (j * SBLK + i, 0)),
        pl.BlockSpec((ROWS, HID), lambda i, j: (i, 0)),
        pl.BlockSpec((ROWS, 1), lambda i, j: (j * SBLK + i, 0)),
        pl.BlockSpec((8, HID), lambda i, j: (0, 0)),
        pl.BlockSpec((HID,), lambda i, j: (0,)),
        pl.BlockSpec((HID,), lambda i, j: (0,)),
    ],
    out_specs=pl.BlockSpec((ROWS, HID), lambda i, j: (j * SBLK + i, 0)),
    out_shape=jax.ShapeDtypeStruct((N, HID), jnp.float32),
)


def kernel(input_ids, token_type_ids, word_embeddings, token_type_embeddings,
           position_embeddings, ln_gamma, ln_beta):
    ids = input_ids.reshape(N).astype(jnp.int32)
    tts = token_type_ids.reshape(N, 1).astype(jnp.float32)
    ttemb = jnp.concatenate(
        [token_type_embeddings,
         jnp.zeros((6, HID), token_type_embeddings.dtype)], axis=0)
    x = _sc_gather(ids, word_embeddings)
    out = _ln_call(x, position_embeddings, tts, ttemb, ln_gamma, ln_beta)
    return out.reshape(BATCH, SEQ, HID)
